# baseline (speedup 1.0000x reference)
import functools

import jax
import jax.numpy as jnp
from jax import lax
from jax.experimental import pallas as pl
from jax.experimental.pallas import tpu as pltpu
from jax.experimental.pallas import tpu_sc as plsc

_NC = 2
_NS = 16
_NW = _NC * _NS
_LANES = 16


def _sc_mean_pool(x, tpair):
    nrows, hist = x.shape
    _, d2 = tpair.shape
    d = d2 // 2
    rows_per_w = nrows // _NW
    nvec = d // _LANES
    inv = 1.0 / float(hist)

    mesh = plsc.VectorSubcoreMesh(core_axis_name="c", subcore_axis_name="s")
    nbuf = 2
    assert rows_per_w % nbuf == 0
    ngrp = hist // _LANES
    tail = hist - _LANES * ngrp
    chunk_starts = list(range(0, hist - _LANES + 1, _LANES))
    if chunk_starts[-1] != hist - _LANES:
        chunk_starts.append(hist - _LANES)

    @functools.partial(
        pl.kernel,
        mesh=mesh,
        out_type=jax.ShapeDtypeStruct((nrows, d), jnp.float32),
        scratch_types=[
            pltpu.VMEM((rows_per_w, hist), jnp.int32),
            [pltpu.VMEM((hist,), jnp.int32)] * nbuf,
            [pltpu.VMEM((hist, d2), jnp.float32)] * nbuf,
            pltpu.VMEM((rows_per_w, d), jnp.float32),
            [pltpu.SemaphoreType.DMA] * nbuf,
        ],
    )
    def pool(x_hbm, tpair_hbm, out_hbm, idx, idxrows, bufs, pooled_v, sems):
        cid = lax.axis_index("c")
        sid = lax.axis_index("s")
        wid = sid * _NC + cid
        base = wid * rows_per_w
        pltpu.sync_copy(x_hbm.at[pl.ds(base, rows_per_w)], idx)

        def gather_row(r, u):
            for c in chunk_starts:
                sl = pl.ds(c, _LANES)
                idxrows[u][sl] = lax.shift_right_logical(idx[r, sl], 1)
            pltpu.async_copy(tpair_hbm.at[idxrows[u]], bufs[u], sems[u])

        for k in range(nbuf):
            gather_row(k, k)

        def group_sum(buf, j0, offv, lo, acc):
            for u in range(lo, _LANES):
                h = offv[u]
                acc = tuple(
                    acc[k] + buf[j0 + u, pl.ds(h + _LANES * k, _LANES)]
                    for k in range(nvec))
            return acc

        def chunk_sum(r, buf, acc):
            def body(g, acc):
                j0 = _LANES * g
                offv = lax.shift_left(idx[r, pl.ds(j0, _LANES)] & 1, 6)
                return group_sum(buf, j0, offv, 0, acc)
            acc = lax.fori_loop(0, ngrp, body, acc)
            if tail:
                j0 = hist - _LANES
                offv = lax.shift_left(idx[r, pl.ds(j0, _LANES)] & 1, 6)
                acc = group_sum(buf, j0, offv, _LANES - tail, acc)
            return acc

        zero = jnp.zeros((_LANES,), jnp.float32)

        def outer(g, carry):
            for u in range(nbuf):
                r = nbuf * g + u
                buf = bufs[u]
                pltpu.make_async_copy(
                    tpair_hbm.at[idxrows[u]], buf, sems[u]).wait()
                acc = chunk_sum(r, buf, (zero,) * nvec)

                @pl.when(r + nbuf < rows_per_w)
                def _():
                    gather_row(r + nbuf, u)

                for k in range(nvec):
                    pooled_v[r, pl.ds(_LANES * k, _LANES)] = acc[k] * inv
            return carry

        lax.fori_loop(0, rows_per_w // nbuf, outer, 0)
        pltpu.sync_copy(pooled_v, out_hbm.at[pl.ds(base, rows_per_w)])

    return pool(x, tpair)


def _tc_head(pooled, w, b):
    bn, d = pooled.shape
    n = w.shape[1]

    def body(p_ref, w_ref, b_ref, o_ref):
        z = jnp.dot(p_ref[...], w_ref[...],
                    preferred_element_type=jnp.float32) + b_ref[...]
        m = jnp.max(z, axis=1, keepdims=True)
        e = jnp.exp(z - m)
        o_ref[...] = e / jnp.sum(e, axis=1, keepdims=True)

    return pl.pallas_call(
        body,
        grid=(1,),
        in_specs=[
            pl.BlockSpec((bn, d), lambda i: (0, 0)),
            pl.BlockSpec((d, n), lambda i: (0, 0)),
            pl.BlockSpec((1, n), lambda i: (0, 0)),
        ],
        out_specs=pl.BlockSpec((bn, n), lambda i: (0, 0)),
        out_shape=jax.ShapeDtypeStruct((bn, n), jnp.float32),
    )(pooled, w, b.reshape(1, n))


def kernel(x, table, W, b):
    v, d = table.shape
    tpair = table.reshape(v // 2, 2 * d)
    pooled = _sc_mean_pool(x, tpair)
    return _tc_head(pooled, W, b)


# speedup vs baseline: 1.0340x; 1.0340x over previous
import functools

import jax
import jax.numpy as jnp
from jax import lax
from jax.experimental import pallas as pl
from jax.experimental.pallas import tpu as pltpu
from jax.experimental.pallas import tpu_sc as plsc

_NC = 2
_NS = 16
_NW = _NC * _NS
_LANES = 16


def _sc_sum_pool(x, tpair, inv):
    nrows, hist = x.shape
    _, d2 = tpair.shape
    d = d2 // 2
    rows_per_w = nrows // _NW
    nvec = d // _LANES

    mesh = plsc.VectorSubcoreMesh(core_axis_name="c", subcore_axis_name="s")
    nbuf = 4
    assert rows_per_w % nbuf == 0
    ngrp = hist // _LANES
    tail = hist - _LANES * ngrp
    chunk_starts = list(range(0, hist - _LANES + 1, _LANES))
    if chunk_starts[-1] != hist - _LANES:
        chunk_starts.append(hist - _LANES)

    @functools.partial(
        pl.kernel,
        mesh=mesh,
        out_type=jax.ShapeDtypeStruct((nrows, d), jnp.float32),
        scratch_types=[
            pltpu.VMEM((rows_per_w, hist), jnp.int32),
            [pltpu.VMEM((hist,), jnp.int32)] * nbuf,
            [pltpu.VMEM((hist, d2), jnp.float32)] * nbuf,
            pltpu.VMEM((rows_per_w, d), jnp.float32),
            [pltpu.SemaphoreType.DMA] * nbuf,
        ],
    )
    def pool(x_hbm, tpair_hbm, out_hbm, idx, idxrows, bufs, pooled_v, sems):
        cid = lax.axis_index("c")
        sid = lax.axis_index("s")
        wid = sid * _NC + cid
        base = wid * rows_per_w
        pltpu.sync_copy(x_hbm.at[pl.ds(base, rows_per_w)], idx)

        def gather_row(r, u):
            for c in chunk_starts:
                sl = pl.ds(c, _LANES)
                idxrows[u][sl] = lax.shift_right_logical(idx[r, sl], 1)
            pltpu.async_copy(tpair_hbm.at[idxrows[u]], bufs[u], sems[u])

        for k in range(nbuf):
            gather_row(k, k)

        def group_sum(buf, j0, offv, lo, acc):
            for u in range(lo, _LANES):
                h = offv[u]
                acc = tuple(
                    acc[k] + buf[j0 + u, pl.ds(h + _LANES * k, _LANES)]
                    for k in range(nvec))
            return acc

        def chunk_sum(r, buf, acc):
            def body(g, acc):
                j0 = _LANES * g
                offv = lax.shift_left(idx[r, pl.ds(j0, _LANES)] & 1, 6)
                return group_sum(buf, j0, offv, 0, acc)
            acc = lax.fori_loop(0, ngrp, body, acc)
            if tail:
                j0 = hist - _LANES
                offv = lax.shift_left(idx[r, pl.ds(j0, _LANES)] & 1, 6)
                acc = group_sum(buf, j0, offv, _LANES - tail, acc)
            return acc

        zero = jnp.zeros((_LANES,), jnp.float32)

        def outer(g, carry):
            for u in range(nbuf):
                r = nbuf * g + u
                buf = bufs[u]
                pltpu.make_async_copy(
                    tpair_hbm.at[idxrows[u]], buf, sems[u]).wait()
                acc = chunk_sum(r, buf, (zero,) * nvec)

                @pl.when(r + nbuf < rows_per_w)
                def _():
                    gather_row(r + nbuf, u)

                for k in range(nvec):
                    pooled_v[r, pl.ds(_LANES * k, _LANES)] = acc[k] * inv
            return carry

        lax.fori_loop(0, rows_per_w // nbuf, outer, 0)
        pltpu.sync_copy(pooled_v, out_hbm.at[pl.ds(base, rows_per_w)])

    return pool(x, tpair)


def _tc_head(halves, w, b):
    bn2, d = halves.shape
    bn = bn2 // 2
    n = w.shape[1]

    def body(p_ref, w_ref, b_ref, o_ref):
        p = p_ref[...].reshape(bn, 2, d).sum(axis=1)
        z = jnp.dot(p, w_ref[...],
                    preferred_element_type=jnp.float32) + b_ref[...]
        m = jnp.max(z, axis=1, keepdims=True)
        e = jnp.exp(z - m)
        o_ref[...] = e / jnp.sum(e, axis=1, keepdims=True)

    return pl.pallas_call(
        body,
        grid=(1,),
        in_specs=[
            pl.BlockSpec((bn2, d), lambda i: (0, 0)),
            pl.BlockSpec((d, n), lambda i: (0, 0)),
            pl.BlockSpec((1, n), lambda i: (0, 0)),
        ],
        out_specs=pl.BlockSpec((bn, n), lambda i: (0, 0)),
        out_shape=jax.ShapeDtypeStruct((bn, n), jnp.float32),
    )(halves, w, b.reshape(1, n))


def kernel(x, table, W, b):
    bn, hist = x.shape
    v, d = table.shape
    tpair = table.reshape(v // 2, 2 * d)
    xh = x.reshape(bn * 2, hist // 2)
    halves = _sc_sum_pool(xh, tpair, 1.0 / float(hist))
    return _tc_head(halves, W, b)
